# Initial kernel scaffold; baseline (speedup 1.0000x reference)
import jax
import jax.numpy as jnp
from jax.experimental import pallas as pl


def kernel(x, x_lens, y_padded, y_lens, W_enc, b_enc, emb, W_join, b_join):
    raise NotImplementedError("write your pallas kernel here")


# speedup vs baseline: 18.3551x; 18.3551x over previous
import jax
import jax.numpy as jnp
from jax.experimental import pallas as pl

N, T, FEAT, C, U, V = 4, 512, 80, 256, 48, 256
BLANK = 0
UP = 56
TB = 64
NEG = -1e30
D_TOT = T + U + 1


def _phase1(x_ref, we_ref, be_ref, oh_ref, ohs_ref, emb_ref, wj_ref, bj_ref,
            lpb_ref, lps_ref):
    xb = x_ref[0]
    enc = jnp.dot(xb, we_ref[...], preferred_element_type=jnp.float32) + be_ref[0]
    oh = oh_ref[0]
    dec = jnp.dot(oh, emb_ref[...], preferred_element_type=jnp.float32)
    joint = jnp.tanh(enc[:, None, :] + dec[None, :, :])
    logits = jnp.dot(joint.reshape(TB * UP, C), wj_ref[...],
                     preferred_element_type=jnp.float32) + bj_ref[0]
    l3 = logits.reshape(TB, UP, V)
    m = jnp.max(l3, axis=2)
    lse = m + jnp.log(jnp.sum(jnp.exp(l3 - m[:, :, None]), axis=2))
    lpb = l3[:, :, BLANK] - lse
    sym = jnp.sum(l3 * ohs_ref[0][None], axis=2) - lse
    ucol = jax.lax.broadcasted_iota(jnp.int32, (TB, UP), 1)
    lpb_ref[0] = jnp.where(ucol <= U, lpb, NEG)
    lps_ref[0] = jnp.where(ucol < U, sym, NEG)


def _phase2(sb_ref, ss_ref, dn_ref, un_ref, out_ref):
    u_iota = jax.lax.broadcasted_iota(jnp.int32, (N, UP), 1)
    dn = dn_ref[...]
    un = un_ref[...]
    a0 = jnp.where(u_iota == 0, 0.0, NEG)
    zeros = jnp.zeros((N, UP), jnp.float32)

    def body(d, carry):
        a, acc_a, acc_b = carry
        sb = sb_ref[d]
        ss = ss_ref[d]
        a_sh = jnp.concatenate(
            [jnp.full((N, 1), NEG, jnp.float32), a[:, :-1]], axis=1)
        t1 = a + sb
        t2 = a_sh + ss
        mx = jnp.maximum(t1, t2)
        mn = jnp.minimum(t1, t2)
        a_new = mx + jnp.log1p(jnp.exp(mn - mx))
        hit_a = (dn == d) & (un == u_iota)
        hit_b = (dn + 1 == d) & (un == u_iota)
        acc_a = jnp.where(hit_a, a_new, acc_a)
        acc_b = jnp.where(hit_b, sb, acc_b)
        return a_new, acc_a, acc_b

    _, acc_a, acc_b = jax.lax.fori_loop(1, D_TOT, body, (a0, zeros, zeros))
    out_ref[...] = -jnp.sum(acc_a + acc_b, axis=(0, 1), keepdims=True)


def _shear(arrT, left_pad, width):
    w = width + left_pad
    p = jnp.pad(arrT, ((0, 0), (0, 0), (left_pad, (w + UP + 1) - w)),
                constant_values=NEG)
    flat = p.reshape(N, UP * (w + UP + 1))[:, :UP * (w + UP)]
    sh = flat.reshape(N, UP, w + UP)[:, :, :D_TOT]
    return jnp.transpose(sh, (2, 0, 1))


def kernel(x, x_lens, y_padded, y_lens, W_enc, b_enc, emb, W_join, b_join):
    f32 = jnp.float32
    sos_y = jnp.concatenate(
        [jnp.zeros((N, 1), y_padded.dtype), y_padded], axis=1)
    sos_pad = jnp.pad(sos_y, ((0, 0), (0, UP - (U + 1))))
    vio = jnp.arange(V, dtype=sos_pad.dtype)
    oh = (sos_pad[:, :, None] == vio).astype(f32)
    yp_pad = jnp.pad(y_padded, ((0, 0), (0, UP - U)), constant_values=-1)
    ohs = (yp_pad[:, :, None] == vio).astype(f32)

    lpb, lps = pl.pallas_call(
        _phase1,
        grid=(N, T // TB),
        in_specs=[
            pl.BlockSpec((1, TB, FEAT), lambda n, t: (n, t, 0)),
            pl.BlockSpec((FEAT, C), lambda n, t: (0, 0)),
            pl.BlockSpec((1, C), lambda n, t: (0, 0)),
            pl.BlockSpec((1, UP, V), lambda n, t: (n, 0, 0)),
            pl.BlockSpec((1, UP, V), lambda n, t: (n, 0, 0)),
            pl.BlockSpec((V, C), lambda n, t: (0, 0)),
            pl.BlockSpec((C, V), lambda n, t: (0, 0)),
            pl.BlockSpec((1, V), lambda n, t: (0, 0)),
        ],
        out_specs=[
            pl.BlockSpec((1, TB, UP), lambda n, t: (n, t, 0)),
            pl.BlockSpec((1, TB, UP), lambda n, t: (n, t, 0)),
        ],
        out_shape=[
            jax.ShapeDtypeStruct((N, T, UP), f32),
            jax.ShapeDtypeStruct((N, T, UP), f32),
        ],
    )(x.astype(f32), W_enc.astype(f32), b_enc.reshape(1, C).astype(f32),
      oh, ohs, emb.astype(f32), W_join.astype(f32),
      b_join.reshape(1, V).astype(f32))

    sb = _shear(jnp.transpose(lpb, (0, 2, 1)), 1, T)
    lpsT = jnp.transpose(lps, (0, 2, 1))
    ls2 = jnp.pad(lpsT, ((0, 0), (1, 0), (0, 0)),
                  constant_values=NEG)[:, :UP]
    ss = _shear(ls2, 0, T)

    dn = (x_lens - 1 + y_lens).astype(jnp.int32)
    un = y_lens.astype(jnp.int32)
    dn_b = jnp.broadcast_to(dn[:, None], (N, UP))
    un_b = jnp.broadcast_to(un[:, None], (N, UP))

    loss = pl.pallas_call(
        _phase2,
        out_shape=jax.ShapeDtypeStruct((1, 1), f32),
    )(sb, ss, dn_b, un_b)
    return loss[0, 0]


# speedup vs baseline: 18.5893x; 1.0128x over previous
import jax
import jax.numpy as jnp
from jax.experimental import pallas as pl
from jax.experimental.pallas import tpu as pltpu

N, T, FEAT, C, U, V = 4, 512, 80, 256, 48, 256
BLANK = 0
UP = 56
TB = 64
NEG = -1e30
D_TOT = T + U + 1


def _phase1(x_ref, we_ref, be_ref, oh_ref, ohs_ref, emb_ref, wj_ref, bj_ref,
            lpb_ref, lps_ref, dec_ref):
    xb = x_ref[0]
    enc = jnp.dot(xb, we_ref[...], preferred_element_type=jnp.float32) + be_ref[0]

    @pl.when(pl.program_id(1) == 0)
    def _():
        oh = oh_ref[0]
        dec_ref[...] = jnp.dot(oh, emb_ref[...],
                               preferred_element_type=jnp.float32)

    dec = dec_ref[...]
    joint = jnp.tanh(enc[:, None, :] + dec[None, :, :])
    logits = jnp.dot(joint.reshape(TB * UP, C), wj_ref[...],
                     preferred_element_type=jnp.float32) + bj_ref[0]
    l3 = logits.reshape(TB, UP, V)
    m = jnp.max(l3, axis=2)
    lse = m + jnp.log(jnp.sum(jnp.exp(l3 - m[:, :, None]), axis=2))
    iota_v = jax.lax.broadcasted_iota(jnp.int32, (TB, UP, V), 2)
    lpb = jnp.sum(jnp.where(iota_v == BLANK, l3, 0.0), axis=2) - lse
    sym = jnp.sum(l3 * ohs_ref[0][None], axis=2) - lse
    ucol = jax.lax.broadcasted_iota(jnp.int32, (TB, UP), 1)
    lpb_ref[0] = jnp.where(ucol <= U, lpb, NEG)
    lps_ref[0] = jnp.where(ucol < U, sym, NEG)


def _phase2(sb_ref, ss_ref, dn_ref, un_ref, out_ref):
    u_iota = jax.lax.broadcasted_iota(jnp.int32, (N, UP), 1)
    dn = dn_ref[...]
    un = un_ref[...]
    a0 = jnp.where(u_iota == 0, 0.0, NEG)
    zeros = jnp.zeros((N, UP), jnp.float32)

    def body(d, carry):
        a, acc_a, acc_b = carry
        sb = sb_ref[d]
        ss = ss_ref[d]
        a_sh = jnp.concatenate(
            [jnp.full((N, 1), NEG, jnp.float32), a[:, :-1]], axis=1)
        t1 = a + sb
        t2 = a_sh + ss
        mx = jnp.maximum(t1, t2)
        mn = jnp.minimum(t1, t2)
        a_new = mx + jnp.log1p(jnp.exp(mn - mx))
        hit_a = (dn == d) & (un == u_iota)
        hit_b = (dn + 1 == d) & (un == u_iota)
        acc_a = jnp.where(hit_a, a_new, acc_a)
        acc_b = jnp.where(hit_b, sb, acc_b)
        return a_new, acc_a, acc_b

    _, acc_a, acc_b = jax.lax.fori_loop(1, D_TOT, body, (a0, zeros, zeros))
    out_ref[...] = -jnp.sum(acc_a + acc_b, axis=(0, 1), keepdims=True)


def _shear(arrT, left_pad, width):
    w = width + left_pad
    p = jnp.pad(arrT, ((0, 0), (0, 0), (left_pad, (w + UP + 1) - w)),
                constant_values=NEG)
    flat = p.reshape(N, UP * (w + UP + 1))[:, :UP * (w + UP)]
    sh = flat.reshape(N, UP, w + UP)[:, :, :D_TOT]
    return jnp.transpose(sh, (2, 0, 1))


def kernel(x, x_lens, y_padded, y_lens, W_enc, b_enc, emb, W_join, b_join):
    f32 = jnp.float32
    sos_y = jnp.concatenate(
        [jnp.zeros((N, 1), y_padded.dtype), y_padded], axis=1)
    sos_pad = jnp.pad(sos_y, ((0, 0), (0, UP - (U + 1))))
    vio = jnp.arange(V, dtype=sos_pad.dtype)
    oh = (sos_pad[:, :, None] == vio).astype(f32)
    yp_pad = jnp.pad(y_padded, ((0, 0), (0, UP - U)), constant_values=-1)
    ohs = (yp_pad[:, :, None] == vio).astype(f32)

    lpb, lps = pl.pallas_call(
        _phase1,
        grid=(N, T // TB),
        in_specs=[
            pl.BlockSpec((1, TB, FEAT), lambda n, t: (n, t, 0)),
            pl.BlockSpec((FEAT, C), lambda n, t: (0, 0)),
            pl.BlockSpec((1, C), lambda n, t: (0, 0)),
            pl.BlockSpec((1, UP, V), lambda n, t: (n, 0, 0)),
            pl.BlockSpec((1, UP, V), lambda n, t: (n, 0, 0)),
            pl.BlockSpec((V, C), lambda n, t: (0, 0)),
            pl.BlockSpec((C, V), lambda n, t: (0, 0)),
            pl.BlockSpec((1, V), lambda n, t: (0, 0)),
        ],
        out_specs=[
            pl.BlockSpec((1, TB, UP), lambda n, t: (n, t, 0)),
            pl.BlockSpec((1, TB, UP), lambda n, t: (n, t, 0)),
        ],
        out_shape=[
            jax.ShapeDtypeStruct((N, T, UP), f32),
            jax.ShapeDtypeStruct((N, T, UP), f32),
        ],
        scratch_shapes=[pltpu.VMEM((UP, C), jnp.float32)],
    )(x.astype(f32), W_enc.astype(f32), b_enc.reshape(1, C).astype(f32),
      oh, ohs, emb.astype(f32), W_join.astype(f32),
      b_join.reshape(1, V).astype(f32))

    sb = _shear(jnp.transpose(lpb, (0, 2, 1)), 1, T)
    lpsT = jnp.transpose(lps, (0, 2, 1))
    ls2 = jnp.pad(lpsT, ((0, 0), (1, 0), (0, 0)),
                  constant_values=NEG)[:, :UP]
    ss = _shear(ls2, 0, T)

    dn = (x_lens - 1 + y_lens).astype(jnp.int32)
    un = y_lens.astype(jnp.int32)
    dn_b = jnp.broadcast_to(dn[:, None], (N, UP))
    un_b = jnp.broadcast_to(un[:, None], (N, UP))

    loss = pl.pallas_call(
        _phase2,
        out_shape=jax.ShapeDtypeStruct((1, 1), f32),
    )(sb, ss, dn_b, un_b)
    return loss[0, 0]
